# baseline (speedup 1.0000x reference)
import jax
import jax.numpy as jnp
from jax.experimental import pallas as pl
from jax.experimental.pallas import tpu as pltpu
from jax.experimental.pallas import tpu_sc as plsc


def kernel(x):
    n, r, c = x.shape
    C2 = 128
    rows_per_slab = r * c // C2
    R2 = n * rows_per_slab
    BR = 128
    nblocks = R2 // BR
    x2 = x.reshape(R2, C2)

    row = jnp.arange(R2, dtype=jnp.int32)
    dest = ((n - 1 - row // rows_per_slab) * rows_per_slab + row % rows_per_slab)
    dest = dest.reshape(1, R2)

    mesh = plsc.VectorSubcoreMesh(core_axis_name="core", subcore_axis_name="subcore")

    @pl.kernel(out_type=jax.ShapeDtypeStruct((R2, C2), x.dtype), mesh=mesh)
    def sc_reverse(x_hbm, d_hbm, o_hbm):
        def body(x_vmem, i_vmem):
            pltpu.sync_copy(x_vmem, o_hbm.at[i_vmem.at[0]])

        pltpu.emit_pipeline(
            body,
            grid=(nblocks,),
            in_specs=[
                pl.BlockSpec((BR, C2), lambda i: (i, 0)),
                pl.BlockSpec((1, BR), lambda i: (0, i)),
            ],
            out_specs=[],
            core_axis_name=("core", "subcore"),
            dimension_semantics=(pltpu.PARALLEL,),
        )(x_hbm, d_hbm)

    return sc_reverse(x2, dest).reshape(n, r, c)


# speedup vs baseline: 1.1882x; 1.1882x over previous
import jax
import jax.numpy as jnp
from jax.experimental import pallas as pl
from jax.experimental.pallas import tpu as pltpu
from jax.experimental.pallas import tpu_sc as plsc


def kernel(x):
    n, r, c = x.shape
    BR = 16
    jb = r // BR
    x2 = x.reshape(n * r, c)

    mesh = plsc.VectorSubcoreMesh(core_axis_name="core", subcore_axis_name="subcore")

    @pl.kernel(out_type=jax.ShapeDtypeStruct((n * r, c), x.dtype), mesh=mesh)
    def sc_reverse(x_hbm, o_hbm):
        def body(in_vmem, out_vmem):
            out_vmem[...] = in_vmem[...]

        pltpu.emit_pipeline(
            body,
            grid=(n, jb),
            in_specs=[pl.BlockSpec((BR, c), lambda i, j: ((n - 1 - i) * jb + j, 0))],
            out_specs=[pl.BlockSpec((BR, c), lambda i, j: (i * jb + j, 0))],
            core_axis_name=("core", "subcore"),
            dimension_semantics=(pltpu.PARALLEL, pltpu.PARALLEL),
        )(x_hbm, o_hbm)

    return sc_reverse(x2).reshape(n, r, c)


# speedup vs baseline: 1.4457x; 1.2167x over previous
import jax
import jax.numpy as jnp
from jax.experimental import pallas as pl
from jax.experimental.pallas import tpu as pltpu
from jax.experimental.pallas import tpu_sc as plsc

_LANES = 16


def kernel(x):
    n, r, c = x.shape
    BR = 32
    jb = r // BR
    x2 = x.reshape(n * r, c)

    mesh = plsc.VectorSubcoreMesh(core_axis_name="core", subcore_axis_name="subcore")

    @pl.kernel(out_type=jax.ShapeDtypeStruct((n * r, c), x.dtype), mesh=mesh)
    def sc_reverse(x_hbm, o_hbm):
        def body(in_vmem, out_vmem):
            @pl.loop(0, BR)
            def _(c0):
                @pl.loop(0, c, step=_LANES, unroll=True)
                def _(c1):
                    slc = (pl.ds(c0, 1), pl.ds(c1, _LANES))
                    out_vmem.at[*slc][...] = in_vmem.at[*slc][...]

        pltpu.emit_pipeline(
            body,
            grid=(n, jb),
            in_specs=[pl.BlockSpec((BR, c), lambda i, j: ((n - 1 - i) * jb + j, 0))],
            out_specs=[pl.BlockSpec((BR, c), lambda i, j: (i * jb + j, 0))],
            core_axis_name=("core", "subcore"),
            dimension_semantics=(pltpu.PARALLEL, pltpu.PARALLEL),
        )(x_hbm, o_hbm)

    return sc_reverse(x2).reshape(n, r, c)


# speedup vs baseline: 4.7355x; 3.2756x over previous
import jax
import jax.numpy as jnp
from jax.experimental import pallas as pl
from jax.experimental.pallas import tpu as pltpu


def _copy_kernel(x_ref, o_ref):
    o_ref[...] = x_ref[...]


def kernel(x):
    n, r, c = x.shape
    return pl.pallas_call(
        _copy_kernel,
        grid=(n,),
        in_specs=[pl.BlockSpec((1, r, c), lambda i: (n - 1 - i, 0, 0))],
        out_specs=pl.BlockSpec((1, r, c), lambda i: (i, 0, 0)),
        out_shape=jax.ShapeDtypeStruct((n, r, c), x.dtype),
        compiler_params=pltpu.CompilerParams(
            dimension_semantics=("parallel",),
        ),
    )(x)
